# Initial kernel scaffold; baseline (speedup 1.0000x reference)
import jax
import jax.numpy as jnp
from jax.experimental import pallas as pl


def kernel(x, edge_index, head_ids, tail_ids, W_self1, W_neigh1, b1, W_self2, W_neigh2, b2, fc_W, fc_b, bn1_g, bn1_b, fc1_W, fc1_b, bn2_g, bn2_b, fc2_W, fc2_b):
    raise NotImplementedError("write your pallas kernel here")


# speedup vs baseline: 7.3809x; 7.3809x over previous
import functools

import jax
import jax.numpy as jnp
from jax import lax
from jax.experimental import pallas as pl
from jax.experimental.pallas import tpu as pltpu
from jax.experimental.pallas import tpu_sc as plsc

N = 10000
D = 128
E = 320000
B = 1024
HID = 256
OUT = 64

NC = 2
NS = 16
NW = NC * NS

RW = 80
ROWS = E // RW
NR = ROWS // NW
NP = 10240
NPS = NP // NS
ZR = 128

XH = 80
NRS = ROWS // NS
SE = 80

@functools.lru_cache(maxsize=None)
def _mesh():
    return plsc.VectorSubcoreMesh(core_axis_name="c", subcore_axis_name="s",
                                  num_cores=NC, num_subcores=NS)


def _zero_shared(zbuf, shared, s, width):
    z16 = jnp.zeros((16,), jnp.float32)

    def zrow(i, carry):
        for k in range(width // 16):
            zbuf[i, pl.ds(k * 16, 16)] = z16
        return carry

    lax.fori_loop(0, ZR, zrow, 0)
    for k in range(NPS // ZR):
        pltpu.sync_copy(zbuf, shared.at[pl.ds(s * NPS + k * ZR, ZR)])


def _sc_agg1_body(xcat_hbm, src_hbm, dst_hbm, out_hbm,
                  sidx, didx, rows, zbuf, shared, sem):
    c = lax.axis_index("c")
    s = lax.axis_index("s")

    pltpu.sync_copy(src_hbm.at[s], sidx)
    pltpu.sync_copy(dst_hbm.at[s], didx)

    off = c * NP

    def adj(i, carry):
        for k in range(RW // 16):
            sidx[i, pl.ds(k * 16, 16)] = sidx[i, pl.ds(k * 16, 16)] + off
        return carry

    lax.fori_loop(0, NRS, adj, 0)

    _zero_shared(zbuf, shared, s, XH)
    plsc.subcore_barrier()

    def body(j, carry):
        pltpu.async_copy(xcat_hbm.at[sidx.at[j]], rows, sem).wait()
        pltpu.sync_copy(rows, shared.at[didx.at[j]], add=True)
        return carry

    lax.fori_loop(0, NRS, body, 0)
    plsc.subcore_barrier()
    pltpu.sync_copy(shared.at[pl.ds(s * NPS, NPS)],
                    out_hbm.at[c, pl.ds(s * NPS, NPS), :])


@functools.lru_cache(maxsize=None)
def _sc_agg1():
    return pl.kernel(
        _sc_agg1_body,
        out_type=jax.ShapeDtypeStruct((NC, NP, XH), jnp.float32),
        mesh=_mesh(),
        compiler_params=pltpu.CompilerParams(use_tc_tiling_on_sc=False),
        scratch_types=[
            pltpu.VMEM((NRS, RW), jnp.int32),
            pltpu.VMEM((NRS, RW), jnp.int32),
            pltpu.VMEM((RW, XH), jnp.float32),
            pltpu.VMEM((ZR, XH), jnp.float32),
            pltpu.VMEM_SHARED((NP, XH), jnp.float32),
            pltpu.SemaphoreType.DMA,
        ],
    )


def _sc_layer2_body(g1_hbm, se_hbm, src_hbm, dst_hbm, ids_hbm,
                    ap_hbm, sg_hbm,
                    sidx, didx, rows, zbuf, idv, serows, idv2, aggrows,
                    shared, sem):
    c = lax.axis_index("c")
    s = lax.axis_index("s")
    wid = s * NC + c

    pltpu.sync_copy(src_hbm.at[wid], sidx)
    pltpu.sync_copy(dst_hbm.at[wid], didx)
    _zero_shared(zbuf, shared, s, OUT)
    plsc.subcore_barrier()

    def body(j, carry):
        pltpu.async_copy(g1_hbm.at[sidx.at[j]], rows, sem).wait()
        pltpu.sync_copy(rows, shared.at[didx.at[j]], add=True)
        return carry

    lax.fori_loop(0, NR, body, 0)

    nid = (2 * B) // NW
    pltpu.sync_copy(ids_hbm.at[pl.ds(wid * nid, nid)], idv)
    pltpu.async_copy(se_hbm.at[idv], serows, sem).wait()
    pltpu.sync_copy(serows, sg_hbm.at[pl.ds(wid * nid, nid)])

    plsc.subcore_barrier()

    nid2 = (2 * B) // NS
    pltpu.sync_copy(ids_hbm.at[pl.ds(s * nid2, nid2)], idv2)
    pltpu.async_copy(shared.at[idv2], aggrows, sem).wait()
    pltpu.sync_copy(aggrows, ap_hbm.at[c, pl.ds(s * nid2, nid2), :])


@functools.lru_cache(maxsize=None)
def _sc_layer2():
    return pl.kernel(
        _sc_layer2_body,
        out_type=[
            jax.ShapeDtypeStruct((NC, 2 * B, OUT), jnp.float32),
            jax.ShapeDtypeStruct((2 * B, SE), jnp.float32),
        ],
        mesh=_mesh(),
        compiler_params=pltpu.CompilerParams(use_tc_tiling_on_sc=False),
        scratch_types=[
            pltpu.VMEM((NR, RW), jnp.int32),
            pltpu.VMEM((NR, RW), jnp.int32),
            pltpu.VMEM((RW, OUT), jnp.float32),
            pltpu.VMEM((ZR, OUT), jnp.float32),
            pltpu.VMEM(((2 * B) // NW,), jnp.int32),
            pltpu.VMEM(((2 * B) // NW, SE), jnp.float32),
            pltpu.VMEM(((2 * B) // NS,), jnp.int32),
            pltpu.VMEM(((2 * B) // NS, OUT), jnp.float32),
            pltpu.VMEM_SHARED((NP, OUT), jnp.float32),
            pltpu.SemaphoreType.DMA,
        ],
    )


_RB = 1024


def _tc_b_kernel(aggp_ref, x_ref, ws1_ref, wn1_ref, b1_ref,
                 ws2_ref, wn2_ref, b2_ref, g1_ref, se_ref):
    a0 = aggp_ref[0]
    a1 = aggp_ref[1]
    deg = jnp.maximum(a1[:, D - XH], 1.0)
    invdeg = 1.0 / deg
    mean = jnp.concatenate([a0, a1[:, :D - XH]], axis=1) * invdeg[:, None]
    h1 = jnp.maximum(
        jnp.dot(x_ref[...], ws1_ref[...], preferred_element_type=jnp.float32)
        + jnp.dot(mean, wn1_ref[...], preferred_element_type=jnp.float32)
        + b1_ref[...][None, :], 0.0)
    g1_ref[...] = jnp.dot(h1, wn2_ref[...], preferred_element_type=jnp.float32)
    sf = jnp.dot(h1, ws2_ref[...], preferred_element_type=jnp.float32) \
        + b2_ref[...][None, :]
    se_ref[...] = jnp.concatenate(
        [sf, invdeg[:, None], jnp.zeros((_RB, SE - OUT - 1), jnp.float32)],
        axis=1)


def _tc_b(aggp, x, ws1, wn1, b1, ws2, wn2, b2):
    return pl.pallas_call(
        _tc_b_kernel,
        grid=(NP // _RB,),
        in_specs=[
            pl.BlockSpec((NC, _RB, XH), lambda i: (0, i, 0)),
            pl.BlockSpec((_RB, D), lambda i: (i, 0)),
            pl.BlockSpec((D, HID), lambda i: (0, 0)),
            pl.BlockSpec((D, HID), lambda i: (0, 0)),
            pl.BlockSpec((HID,), lambda i: (0,)),
            pl.BlockSpec((HID, OUT), lambda i: (0, 0)),
            pl.BlockSpec((HID, OUT), lambda i: (0, 0)),
            pl.BlockSpec((OUT,), lambda i: (0,)),
        ],
        out_specs=[
            pl.BlockSpec((_RB, OUT), lambda i: (i, 0)),
            pl.BlockSpec((_RB, SE), lambda i: (i, 0)),
        ],
        out_shape=[
            jax.ShapeDtypeStruct((NP, OUT), jnp.float32),
            jax.ShapeDtypeStruct((NP, SE), jnp.float32),
        ],
    )(aggp, x, ws1, wn1, b1, ws2, wn2, b2)


def _tc_d_kernel(sg_ref, ap_ref, fcw_ref, fcb_ref, g1_ref, be1_ref,
                 f1w_ref, f1b_ref, g2_ref, be2_ref, f2w_ref, f2b_ref,
                 out_ref):
    agg2 = ap_ref[0] + ap_ref[1]
    slg = sg_ref[:, :OUT] + sg_ref[:, OUT:OUT + 1] * agg2
    fused = jnp.concatenate([slg[:B], slg[B:]], axis=1)

    def bn_relu(h, g, beta):
        mu = jnp.mean(h, axis=0, keepdims=True)
        var = jnp.mean((h - mu) * (h - mu), axis=0, keepdims=True)
        return jnp.maximum(g[None, :] * (h - mu) / jnp.sqrt(var + 1e-5)
                           + beta[None, :], 0.0)

    h = jnp.dot(fused, fcw_ref[...], preferred_element_type=jnp.float32) \
        + fcb_ref[...][None, :]
    h = bn_relu(h, g1_ref[...], be1_ref[...])
    h = jnp.dot(h, f1w_ref[...], preferred_element_type=jnp.float32) \
        + f1b_ref[...][None, :]
    h = bn_relu(h, g2_ref[...], be2_ref[...])
    out_ref[...] = jnp.dot(h, f2w_ref[...],
                           preferred_element_type=jnp.float32) \
        + f2b_ref[...][None, :]


def _tc_d(sg, ap, fc_W, fc_b, bn1_g, bn1_b, fc1_W, fc1_b, bn2_g, bn2_b,
          fc2_W, fc2_b):
    return pl.pallas_call(
        _tc_d_kernel,
        out_shape=jax.ShapeDtypeStruct((B, 1), jnp.float32),
    )(sg, ap, fc_W, fc_b, bn1_g, bn1_b, fc1_W, fc1_b, bn2_g, bn2_b,
      fc2_W, fc2_b)


def kernel(x, edge_index, head_ids, tail_ids,
           W_self1, W_neigh1, b1, W_self2, W_neigh2, b2,
           fc_W, fc_b, bn1_g, bn1_b, fc1_W, fc1_b, bn2_g, bn2_b,
           fc2_W, fc2_b):
    srcf = edge_index[0].astype(jnp.int32)
    dstf = edge_index[1].astype(jnp.int32)
    src = srcf.reshape(NW, NR, RW)
    dst = dstf.reshape(NW, NR, RW)
    src16 = srcf.reshape(NS, NRS, RW)
    dst16 = dstf.reshape(NS, NRS, RW)
    x0 = jnp.pad(x[:, :XH], ((0, NP - N), (0, 0)))
    x1 = jnp.pad(
        jnp.concatenate([x[:, XH:], jnp.ones((N, 1), jnp.float32)], axis=1),
        ((0, NP - N), (0, XH - (D - XH) - 1)))
    xcat = jnp.concatenate([x0, x1], axis=0)
    xp = jnp.pad(x, ((0, NP - N), (0, 0)))
    ids2 = jnp.concatenate([head_ids, tail_ids]).astype(jnp.int32)

    aggp = _sc_agg1()(xcat, src16, dst16)
    g1, se = _tc_b(aggp, xp, W_self1, W_neigh1, b1, W_self2, W_neigh2, b2)
    ap, sg = _sc_layer2()(g1, se, src, dst, ids2)
    return _tc_d(sg, ap, fc_W, fc_b, bn1_g, bn1_b, fc1_W, fc1_b,
                 bn2_g, bn2_b, fc2_W, fc2_b)


# speedup vs baseline: 9.0104x; 1.2208x over previous
import functools

import jax
import jax.numpy as jnp
from jax import lax
from jax.experimental import pallas as pl
from jax.experimental.pallas import tpu as pltpu
from jax.experimental.pallas import tpu_sc as plsc

N = 10000
D = 128
E = 320000
B = 1024
HID = 256
OUT = 64

NC = 2
NS = 16
NW = NC * NS

RW = 80
ROWS = E // RW
NR = ROWS // NW
NP = 10240
NPS = NP // NS
ZR = 128

XH = 80
NRS = ROWS // NS
SE = 80

@functools.lru_cache(maxsize=None)
def _mesh():
    return plsc.VectorSubcoreMesh(core_axis_name="c", subcore_axis_name="s",
                                  num_cores=NC, num_subcores=NS)


def _zero_shared(zbuf, shared, s, width):
    z16 = jnp.zeros((16,), jnp.float32)

    def zrow(i, carry):
        for k in range(width // 16):
            zbuf[i, pl.ds(k * 16, 16)] = z16
        return carry

    lax.fori_loop(0, ZR, zrow, 0)
    for k in range(NPS // ZR):
        pltpu.sync_copy(zbuf, shared.at[pl.ds(s * NPS + k * ZR, ZR)])


def _edge_pipeline(table_hbm, sidx, didx, shared, buf0, buf1, sem0, sem1, n):
    def _start(j, buf, sem):
        pltpu.async_copy(table_hbm.at[sidx.at[j]], buf, sem)

    def _wait(j, buf, sem):
        pltpu.make_async_copy(table_hbm.at[sidx.at[j]], buf, sem).wait()

    def _scat(j, buf):
        pltpu.sync_copy(buf, shared.at[didx.at[j]], add=True)

    _start(0, buf0, sem0)

    def body(j2, carry):
        j = 2 * j2
        _wait(j, buf0, sem0)
        _start(j + 1, buf1, sem1)
        _scat(j, buf0)
        _wait(j + 1, buf1, sem1)
        _start(j + 2, buf0, sem0)
        _scat(j + 1, buf1)
        return carry

    if n % 2 == 0:
        lax.fori_loop(0, (n - 2) // 2, body, 0)
        _wait(n - 2, buf0, sem0)
        _start(n - 1, buf1, sem1)
        _scat(n - 2, buf0)
        _wait(n - 1, buf1, sem1)
        _scat(n - 1, buf1)
    else:
        lax.fori_loop(0, (n - 1) // 2, body, 0)
        _wait(n - 1, buf0, sem0)
        _scat(n - 1, buf0)


def _sc_agg1_body(xcat_hbm, src_hbm, dst_hbm, out_hbm,
                  sidx, didx, rows, rows2, zbuf, shared, sem, sem2):
    c = lax.axis_index("c")
    s = lax.axis_index("s")

    pltpu.sync_copy(src_hbm.at[s], sidx)
    pltpu.sync_copy(dst_hbm.at[s], didx)

    off = c * NP

    def adj(i, carry):
        for k in range(RW // 16):
            sidx[i, pl.ds(k * 16, 16)] = sidx[i, pl.ds(k * 16, 16)] + off
        return carry

    lax.fori_loop(0, NRS, adj, 0)

    _zero_shared(zbuf, shared, s, XH)
    plsc.subcore_barrier()
    _edge_pipeline(xcat_hbm, sidx, didx, shared, rows, rows2, sem, sem2, NRS)
    plsc.subcore_barrier()
    pltpu.sync_copy(shared.at[pl.ds(s * NPS, NPS)],
                    out_hbm.at[c, pl.ds(s * NPS, NPS), :])


@functools.lru_cache(maxsize=None)
def _sc_agg1():
    return pl.kernel(
        _sc_agg1_body,
        out_type=jax.ShapeDtypeStruct((NC, NP, XH), jnp.float32),
        mesh=_mesh(),
        compiler_params=pltpu.CompilerParams(use_tc_tiling_on_sc=False),
        scratch_types=[
            pltpu.VMEM((NRS, RW), jnp.int32),
            pltpu.VMEM((NRS, RW), jnp.int32),
            pltpu.VMEM((RW, XH), jnp.float32),
            pltpu.VMEM((RW, XH), jnp.float32),
            pltpu.VMEM((ZR, XH), jnp.float32),
            pltpu.VMEM_SHARED((NP, XH), jnp.float32),
            pltpu.SemaphoreType.DMA,
            pltpu.SemaphoreType.DMA,
        ],
    )


def _sc_layer2_body(g1_hbm, se_hbm, src_hbm, dst_hbm, ids_hbm,
                    ap_hbm, sg_hbm,
                    sidx, didx, rows, rows2, zbuf, idv, serows, idv2, aggrows,
                    shared, sem, sem2):
    c = lax.axis_index("c")
    s = lax.axis_index("s")
    wid = s * NC + c

    pltpu.sync_copy(src_hbm.at[wid], sidx)
    pltpu.sync_copy(dst_hbm.at[wid], didx)
    _zero_shared(zbuf, shared, s, OUT)
    plsc.subcore_barrier()
    _edge_pipeline(g1_hbm, sidx, didx, shared, rows, rows2, sem, sem2, NR)

    nid = (2 * B) // NW
    pltpu.sync_copy(ids_hbm.at[pl.ds(wid * nid, nid)], idv)
    pltpu.async_copy(se_hbm.at[idv], serows, sem).wait()
    pltpu.sync_copy(serows, sg_hbm.at[pl.ds(wid * nid, nid)])

    plsc.subcore_barrier()

    nid2 = (2 * B) // NS
    pltpu.sync_copy(ids_hbm.at[pl.ds(s * nid2, nid2)], idv2)
    pltpu.async_copy(shared.at[idv2], aggrows, sem).wait()
    pltpu.sync_copy(aggrows, ap_hbm.at[c, pl.ds(s * nid2, nid2), :])


@functools.lru_cache(maxsize=None)
def _sc_layer2():
    return pl.kernel(
        _sc_layer2_body,
        out_type=[
            jax.ShapeDtypeStruct((NC, 2 * B, OUT), jnp.float32),
            jax.ShapeDtypeStruct((2 * B, SE), jnp.float32),
        ],
        mesh=_mesh(),
        compiler_params=pltpu.CompilerParams(use_tc_tiling_on_sc=False),
        scratch_types=[
            pltpu.VMEM((NR, RW), jnp.int32),
            pltpu.VMEM((NR, RW), jnp.int32),
            pltpu.VMEM((RW, OUT), jnp.float32),
            pltpu.VMEM((RW, OUT), jnp.float32),
            pltpu.VMEM((ZR, OUT), jnp.float32),
            pltpu.VMEM(((2 * B) // NW,), jnp.int32),
            pltpu.VMEM(((2 * B) // NW, SE), jnp.float32),
            pltpu.VMEM(((2 * B) // NS,), jnp.int32),
            pltpu.VMEM(((2 * B) // NS, OUT), jnp.float32),
            pltpu.VMEM_SHARED((NP, OUT), jnp.float32),
            pltpu.SemaphoreType.DMA,
            pltpu.SemaphoreType.DMA,
        ],
    )


_RB = 1024


def _tc_b_kernel(aggp_ref, x_ref, ws1_ref, wn1_ref, b1_ref,
                 ws2_ref, wn2_ref, b2_ref, g1_ref, se_ref):
    a0 = aggp_ref[0]
    a1 = aggp_ref[1]
    deg = jnp.maximum(a1[:, D - XH], 1.0)
    invdeg = 1.0 / deg
    mean = jnp.concatenate([a0, a1[:, :D - XH]], axis=1) * invdeg[:, None]
    h1 = jnp.maximum(
        jnp.dot(x_ref[...], ws1_ref[...], preferred_element_type=jnp.float32)
        + jnp.dot(mean, wn1_ref[...], preferred_element_type=jnp.float32)
        + b1_ref[...][None, :], 0.0)
    g1_ref[...] = jnp.dot(h1, wn2_ref[...], preferred_element_type=jnp.float32)
    sf = jnp.dot(h1, ws2_ref[...], preferred_element_type=jnp.float32) \
        + b2_ref[...][None, :]
    se_ref[...] = jnp.concatenate(
        [sf, invdeg[:, None], jnp.zeros((_RB, SE - OUT - 1), jnp.float32)],
        axis=1)


def _tc_b(aggp, x, ws1, wn1, b1, ws2, wn2, b2):
    return pl.pallas_call(
        _tc_b_kernel,
        grid=(NP // _RB,),
        in_specs=[
            pl.BlockSpec((NC, _RB, XH), lambda i: (0, i, 0)),
            pl.BlockSpec((_RB, D), lambda i: (i, 0)),
            pl.BlockSpec((D, HID), lambda i: (0, 0)),
            pl.BlockSpec((D, HID), lambda i: (0, 0)),
            pl.BlockSpec((HID,), lambda i: (0,)),
            pl.BlockSpec((HID, OUT), lambda i: (0, 0)),
            pl.BlockSpec((HID, OUT), lambda i: (0, 0)),
            pl.BlockSpec((OUT,), lambda i: (0,)),
        ],
        out_specs=[
            pl.BlockSpec((_RB, OUT), lambda i: (i, 0)),
            pl.BlockSpec((_RB, SE), lambda i: (i, 0)),
        ],
        out_shape=[
            jax.ShapeDtypeStruct((NP, OUT), jnp.float32),
            jax.ShapeDtypeStruct((NP, SE), jnp.float32),
        ],
    )(aggp, x, ws1, wn1, b1, ws2, wn2, b2)


def _tc_d_kernel(sg_ref, ap_ref, fcw_ref, fcb_ref, g1_ref, be1_ref,
                 f1w_ref, f1b_ref, g2_ref, be2_ref, f2w_ref, f2b_ref,
                 out_ref):
    agg2 = ap_ref[0] + ap_ref[1]
    slg = sg_ref[:, :OUT] + sg_ref[:, OUT:OUT + 1] * agg2
    fused = jnp.concatenate([slg[:B], slg[B:]], axis=1)

    def bn_relu(h, g, beta):
        mu = jnp.mean(h, axis=0, keepdims=True)
        var = jnp.mean((h - mu) * (h - mu), axis=0, keepdims=True)
        return jnp.maximum(g[None, :] * (h - mu) / jnp.sqrt(var + 1e-5)
                           + beta[None, :], 0.0)

    h = jnp.dot(fused, fcw_ref[...], preferred_element_type=jnp.float32) \
        + fcb_ref[...][None, :]
    h = bn_relu(h, g1_ref[...], be1_ref[...])
    h = jnp.dot(h, f1w_ref[...], preferred_element_type=jnp.float32) \
        + f1b_ref[...][None, :]
    h = bn_relu(h, g2_ref[...], be2_ref[...])
    out_ref[...] = jnp.dot(h, f2w_ref[...],
                           preferred_element_type=jnp.float32) \
        + f2b_ref[...][None, :]


def _tc_d(sg, ap, fc_W, fc_b, bn1_g, bn1_b, fc1_W, fc1_b, bn2_g, bn2_b,
          fc2_W, fc2_b):
    return pl.pallas_call(
        _tc_d_kernel,
        out_shape=jax.ShapeDtypeStruct((B, 1), jnp.float32),
    )(sg, ap, fc_W, fc_b, bn1_g, bn1_b, fc1_W, fc1_b, bn2_g, bn2_b,
      fc2_W, fc2_b)


def kernel(x, edge_index, head_ids, tail_ids,
           W_self1, W_neigh1, b1, W_self2, W_neigh2, b2,
           fc_W, fc_b, bn1_g, bn1_b, fc1_W, fc1_b, bn2_g, bn2_b,
           fc2_W, fc2_b):
    srcf = edge_index[0].astype(jnp.int32)
    dstf = edge_index[1].astype(jnp.int32)
    src = srcf.reshape(NW, NR, RW)
    dst = dstf.reshape(NW, NR, RW)
    src16 = srcf.reshape(NS, NRS, RW)
    dst16 = dstf.reshape(NS, NRS, RW)
    x0 = jnp.pad(x[:, :XH], ((0, NP - N), (0, 0)))
    x1 = jnp.pad(
        jnp.concatenate([x[:, XH:], jnp.ones((N, 1), jnp.float32)], axis=1),
        ((0, NP - N), (0, XH - (D - XH) - 1)))
    xcat = jnp.concatenate([x0, x1], axis=0)
    xp = jnp.pad(x, ((0, NP - N), (0, 0)))
    ids2 = jnp.concatenate([head_ids, tail_ids]).astype(jnp.int32)

    aggp = _sc_agg1()(xcat, src16, dst16)
    g1, se = _tc_b(aggp, xp, W_self1, W_neigh1, b1, W_self2, W_neigh2, b2)
    ap, sg = _sc_layer2()(g1, se, src, dst, ids2)
    return _tc_d(sg, ap, fc_W, fc_b, bn1_g, bn1_b, fc1_W, fc1_b,
                 bn2_g, bn2_b, fc2_W, fc2_b)


# speedup vs baseline: 9.4820x; 1.0523x over previous
import functools

import jax
import jax.numpy as jnp
from jax import lax
from jax.experimental import pallas as pl
from jax.experimental.pallas import tpu as pltpu
from jax.experimental.pallas import tpu_sc as plsc

N = 10000
D = 128
E = 320000
B = 1024
HID = 256
OUT = 64

NC = 2
NS = 16
NW = NC * NS

RW = 80
ROWS = E // RW
NR = ROWS // NW
NP = 10240
NPS = NP // NS
ZR = 128

XH = 64
NRS = ROWS // NS
SE = 80

@functools.lru_cache(maxsize=None)
def _mesh():
    return plsc.VectorSubcoreMesh(core_axis_name="c", subcore_axis_name="s",
                                  num_cores=NC, num_subcores=NS)


def _zero_shared(zbuf, shared, s, width):
    z16 = jnp.zeros((16,), jnp.float32)

    def zrow(i, carry):
        for k in range(width // 16):
            zbuf[i, pl.ds(k * 16, 16)] = z16
        return carry

    lax.fori_loop(0, ZR, zrow, 0)
    for k in range(NPS // ZR):
        pltpu.sync_copy(zbuf, shared.at[pl.ds(s * NPS + k * ZR, ZR)])


def _edge_pipeline(table_hbm, sidx, didx, shared, buf0, buf1, sem0, sem1, n,
                   extra=None):
    def _start(j, buf, sem):
        pltpu.async_copy(table_hbm.at[sidx.at[j]], buf, sem)

    def _wait(j, buf, sem):
        pltpu.make_async_copy(table_hbm.at[sidx.at[j]], buf, sem).wait()

    def _scat(j, buf):
        pltpu.sync_copy(buf, shared.at[didx.at[j]], add=True)

    def _x(j):
        if extra is not None:
            extra(j)

    _start(0, buf0, sem0)

    def body(j2, carry):
        j = 2 * j2
        _wait(j, buf0, sem0)
        _start(j + 1, buf1, sem1)
        _x(j)
        _scat(j, buf0)
        _wait(j + 1, buf1, sem1)
        _start(j + 2, buf0, sem0)
        _x(j + 1)
        _scat(j + 1, buf1)
        return carry

    if n % 2 == 0:
        lax.fori_loop(0, (n - 2) // 2, body, 0)
        _wait(n - 2, buf0, sem0)
        _start(n - 1, buf1, sem1)
        _x(n - 2)
        _scat(n - 2, buf0)
        _wait(n - 1, buf1, sem1)
        _x(n - 1)
        _scat(n - 1, buf1)
    else:
        lax.fori_loop(0, (n - 1) // 2, body, 0)
        _wait(n - 1, buf0, sem0)
        _x(n - 1)
        _scat(n - 1, buf0)


def _sc_agg1_body(xcat_hbm, src_hbm, dst_hbm, out_hbm, degp_hbm,
                  sidx, didx, rows, rows2, zbuf, deg, shared, sem, sem2):
    c = lax.axis_index("c")
    s = lax.axis_index("s")
    wid = s * NC + c

    pltpu.sync_copy(src_hbm.at[s], sidx)
    pltpu.sync_copy(dst_hbm.at[s], didx)

    off = c * NP

    def adj(i, carry):
        for k in range(RW // 16):
            sidx[i, pl.ds(k * 16, 16)] = sidx[i, pl.ds(k * 16, 16)] + off
        return carry

    lax.fori_loop(0, NRS, adj, 0)

    z16 = jnp.zeros((16,), jnp.float32)

    def dz(i, carry):
        deg[pl.ds(i * 16, 16)] = z16
        return carry

    lax.fori_loop(0, NP // 16, dz, 0)

    one16 = jnp.ones((16,), jnp.float32)

    def count_deg(j):
        for k in range(RW // 16):
            plsc.addupdate_scatter(deg, [didx[j, pl.ds(k * 16, 16)]], one16)

    _zero_shared(zbuf, shared, s, XH)
    plsc.subcore_barrier()
    _edge_pipeline(xcat_hbm, sidx, didx, shared, rows, rows2, sem, sem2, NRS,
                   extra=count_deg)
    plsc.subcore_barrier()
    pltpu.sync_copy(shared.at[pl.ds(s * NPS, NPS)],
                    out_hbm.at[c, pl.ds(s * NPS, NPS), :])
    pltpu.sync_copy(deg, degp_hbm.at[wid])


@functools.lru_cache(maxsize=None)
def _sc_agg1():
    return pl.kernel(
        _sc_agg1_body,
        out_type=[
            jax.ShapeDtypeStruct((NC, NP, XH), jnp.float32),
            jax.ShapeDtypeStruct((NW, NP), jnp.float32),
        ],
        mesh=_mesh(),
        compiler_params=pltpu.CompilerParams(use_tc_tiling_on_sc=False,
                                             needs_layout_passes=False),
        scratch_types=[
            pltpu.VMEM((NRS, RW), jnp.int32),
            pltpu.VMEM((NRS, RW), jnp.int32),
            pltpu.VMEM((RW, XH), jnp.float32),
            pltpu.VMEM((RW, XH), jnp.float32),
            pltpu.VMEM((ZR, XH), jnp.float32),
            pltpu.VMEM((NP,), jnp.float32),
            pltpu.VMEM_SHARED((NP, XH), jnp.float32),
            pltpu.SemaphoreType.DMA,
            pltpu.SemaphoreType.DMA,
        ],
    )


def _sc_layer2_body(g1_hbm, se_hbm, src_hbm, dst_hbm, ids_hbm,
                    ap_hbm, sg_hbm,
                    sidx, didx, rows, rows2, zbuf, idv, serows, idv2, aggrows,
                    shared, sem, sem2):
    c = lax.axis_index("c")
    s = lax.axis_index("s")
    wid = s * NC + c

    pltpu.sync_copy(src_hbm.at[wid], sidx)
    pltpu.sync_copy(dst_hbm.at[wid], didx)
    _zero_shared(zbuf, shared, s, OUT)
    plsc.subcore_barrier()
    _edge_pipeline(g1_hbm, sidx, didx, shared, rows, rows2, sem, sem2, NR)

    nid = (2 * B) // NW
    pltpu.sync_copy(ids_hbm.at[pl.ds(wid * nid, nid)], idv)
    pltpu.async_copy(se_hbm.at[idv], serows, sem).wait()
    pltpu.sync_copy(serows, sg_hbm.at[pl.ds(wid * nid, nid)])

    plsc.subcore_barrier()

    nid2 = (2 * B) // NS
    pltpu.sync_copy(ids_hbm.at[pl.ds(s * nid2, nid2)], idv2)
    pltpu.async_copy(shared.at[idv2], aggrows, sem).wait()
    pltpu.sync_copy(aggrows, ap_hbm.at[c, pl.ds(s * nid2, nid2), :])


@functools.lru_cache(maxsize=None)
def _sc_layer2():
    return pl.kernel(
        _sc_layer2_body,
        out_type=[
            jax.ShapeDtypeStruct((NC, 2 * B, OUT), jnp.float32),
            jax.ShapeDtypeStruct((2 * B, SE), jnp.float32),
        ],
        mesh=_mesh(),
        compiler_params=pltpu.CompilerParams(use_tc_tiling_on_sc=False,
                                             needs_layout_passes=False),
        scratch_types=[
            pltpu.VMEM((NR, RW), jnp.int32),
            pltpu.VMEM((NR, RW), jnp.int32),
            pltpu.VMEM((RW, OUT), jnp.float32),
            pltpu.VMEM((RW, OUT), jnp.float32),
            pltpu.VMEM((ZR, OUT), jnp.float32),
            pltpu.VMEM(((2 * B) // NW,), jnp.int32),
            pltpu.VMEM(((2 * B) // NW, SE), jnp.float32),
            pltpu.VMEM(((2 * B) // NS,), jnp.int32),
            pltpu.VMEM(((2 * B) // NS, OUT), jnp.float32),
            pltpu.VMEM_SHARED((NP, OUT), jnp.float32),
            pltpu.SemaphoreType.DMA,
            pltpu.SemaphoreType.DMA,
        ],
    )


_RB = 1024


def _tc_b_kernel(aggp_ref, degp_ref, x_ref, ws1_ref, wn1_ref, b1_ref,
                 ws2_ref, wn2_ref, b2_ref, g1_ref, se_ref):
    a0 = aggp_ref[0]
    a1 = aggp_ref[1]
    deg = jnp.maximum(0.5 * jnp.sum(degp_ref[...], axis=0), 1.0)
    invdeg = 1.0 / deg
    mean = jnp.concatenate([a0, a1], axis=1) * invdeg[:, None]
    h1 = jnp.maximum(
        jnp.dot(x_ref[...], ws1_ref[...], preferred_element_type=jnp.float32)
        + jnp.dot(mean, wn1_ref[...], preferred_element_type=jnp.float32)
        + b1_ref[...][None, :], 0.0)
    g1_ref[...] = jnp.dot(h1, wn2_ref[...], preferred_element_type=jnp.float32)
    sf = jnp.dot(h1, ws2_ref[...], preferred_element_type=jnp.float32) \
        + b2_ref[...][None, :]
    se_ref[...] = jnp.concatenate(
        [sf, invdeg[:, None], jnp.zeros((_RB, SE - OUT - 1), jnp.float32)],
        axis=1)


def _tc_b(aggp, degp, x, ws1, wn1, b1, ws2, wn2, b2):
    return pl.pallas_call(
        _tc_b_kernel,
        grid=(NP // _RB,),
        in_specs=[
            pl.BlockSpec((NC, _RB, XH), lambda i: (0, i, 0)),
            pl.BlockSpec((NW, _RB), lambda i: (0, i)),
            pl.BlockSpec((_RB, D), lambda i: (i, 0)),
            pl.BlockSpec((D, HID), lambda i: (0, 0)),
            pl.BlockSpec((D, HID), lambda i: (0, 0)),
            pl.BlockSpec((HID,), lambda i: (0,)),
            pl.BlockSpec((HID, OUT), lambda i: (0, 0)),
            pl.BlockSpec((HID, OUT), lambda i: (0, 0)),
            pl.BlockSpec((OUT,), lambda i: (0,)),
        ],
        out_specs=[
            pl.BlockSpec((_RB, OUT), lambda i: (i, 0)),
            pl.BlockSpec((_RB, SE), lambda i: (i, 0)),
        ],
        out_shape=[
            jax.ShapeDtypeStruct((NP, OUT), jnp.float32),
            jax.ShapeDtypeStruct((NP, SE), jnp.float32),
        ],
    )(aggp, degp, x, ws1, wn1, b1, ws2, wn2, b2)


def _tc_d_kernel(sg_ref, ap_ref, fcw_ref, fcb_ref, g1_ref, be1_ref,
                 f1w_ref, f1b_ref, g2_ref, be2_ref, f2w_ref, f2b_ref,
                 out_ref):
    agg2 = ap_ref[0] + ap_ref[1]
    slg = sg_ref[:, :OUT] + sg_ref[:, OUT:OUT + 1] * agg2
    fused = jnp.concatenate([slg[:B], slg[B:]], axis=1)

    def bn_relu(h, g, beta):
        mu = jnp.mean(h, axis=0, keepdims=True)
        var = jnp.mean((h - mu) * (h - mu), axis=0, keepdims=True)
        return jnp.maximum(g[None, :] * (h - mu) / jnp.sqrt(var + 1e-5)
                           + beta[None, :], 0.0)

    h = jnp.dot(fused, fcw_ref[...], preferred_element_type=jnp.float32) \
        + fcb_ref[...][None, :]
    h = bn_relu(h, g1_ref[...], be1_ref[...])
    h = jnp.dot(h, f1w_ref[...], preferred_element_type=jnp.float32) \
        + f1b_ref[...][None, :]
    h = bn_relu(h, g2_ref[...], be2_ref[...])
    out_ref[...] = jnp.dot(h, f2w_ref[...],
                           preferred_element_type=jnp.float32) \
        + f2b_ref[...][None, :]


def _tc_d(sg, ap, fc_W, fc_b, bn1_g, bn1_b, fc1_W, fc1_b, bn2_g, bn2_b,
          fc2_W, fc2_b):
    return pl.pallas_call(
        _tc_d_kernel,
        out_shape=jax.ShapeDtypeStruct((B, 1), jnp.float32),
    )(sg, ap, fc_W, fc_b, bn1_g, bn1_b, fc1_W, fc1_b, bn2_g, bn2_b,
      fc2_W, fc2_b)


def kernel(x, edge_index, head_ids, tail_ids,
           W_self1, W_neigh1, b1, W_self2, W_neigh2, b2,
           fc_W, fc_b, bn1_g, bn1_b, fc1_W, fc1_b, bn2_g, bn2_b,
           fc2_W, fc2_b):
    srcf = edge_index[0].astype(jnp.int32)
    dstf = edge_index[1].astype(jnp.int32)
    src = srcf.reshape(NW, NR, RW)
    dst = dstf.reshape(NW, NR, RW)
    src16 = srcf.reshape(NS, NRS, RW)
    dst16 = dstf.reshape(NS, NRS, RW)
    xp = jnp.pad(x, ((0, NP - N), (0, 0)))
    xcat = jnp.concatenate([xp[:, :XH], xp[:, XH:]], axis=0)
    ids2 = jnp.concatenate([head_ids, tail_ids]).astype(jnp.int32)

    aggp, degp = _sc_agg1()(xcat, src16, dst16)
    g1, se = _tc_b(aggp, degp, xp, W_self1, W_neigh1, b1, W_self2, W_neigh2,
                   b2)
    ap, sg = _sc_layer2()(g1, se, src, dst, ids2)
    return _tc_d(sg, ap, fc_W, fc_b, bn1_g, bn1_b, fc1_W, fc1_b,
                 bn2_g, bn2_b, fc2_W, fc2_b)


# speedup vs baseline: 13.3954x; 1.4127x over previous
import functools

import jax
import jax.numpy as jnp
from jax import lax
from jax.experimental import pallas as pl
from jax.experimental.pallas import tpu as pltpu
from jax.experimental.pallas import tpu_sc as plsc

N = 10000
D = 128
E = 320000
B = 1024
HID = 256
OUT = 64

NC = 2
NS = 16
NW = NC * NS

RW = 80
ROWS = E // RW
NR = ROWS // NW
NP = 10240
NPS = NP // NS
ZR = 128

XH = 64
NRS = ROWS // NS
SE = 80

@functools.lru_cache(maxsize=None)
def _mesh():
    return plsc.VectorSubcoreMesh(core_axis_name="c", subcore_axis_name="s",
                                  num_cores=NC, num_subcores=NS)


def _zero_shared(zbuf, shared, s, width):
    z16 = jnp.zeros((16,), jnp.float32)

    def zrow(i, carry):
        for k in range(width // 16):
            zbuf[i, pl.ds(k * 16, 16)] = z16
        return carry

    lax.fori_loop(0, ZR, zrow, 0)
    for k in range(NPS // ZR):
        pltpu.sync_copy(zbuf, shared.at[pl.ds(s * NPS + k * ZR, ZR)])


def _edge_pipeline(table_hbm, sidx, didx, shared, bufs, gsems, ssems, n,
                   extra=None):
    assert n >= 6

    def startG(r, k):
        pltpu.async_copy(table_hbm.at[sidx.at[r]], bufs[k], gsems[k])

    def waitG(r, k):
        pltpu.make_async_copy(table_hbm.at[sidx.at[r]], bufs[k],
                              gsems[k]).wait()

    def startS(r, k):
        pltpu.async_copy(bufs[k], shared.at[didx.at[r]], ssems[k], add=True)

    def waitS(r, k):
        pltpu.make_async_copy(bufs[k], shared.at[didx.at[r]],
                              ssems[k]).wait()

    def slot(r, k, has_prev2, has_next2):
        waitG(r, k)
        startS(r, k)
        if has_prev2:
            waitS(r - 2, (k - 2) % 4)
        if has_next2:
            startG(r + 2, (k + 2) % 4)
        if extra is not None:
            extra(r)

    startG(0, 0)
    startG(1, 1)
    for r in range(4):
        slot(r, r, r >= 2, r + 2 < n)

    M = (n - 6) // 4

    def body(m, carry):
        base = 4 + 4 * m
        for k in range(4):
            slot(base + k, k, True, True)
        return carry

    lax.fori_loop(0, M, body, 0)
    for r in range(4 + 4 * M, n):
        slot(r, r % 4, True, r + 2 < n)
    waitS(n - 2, (n - 2) % 4)
    waitS(n - 1, (n - 1) % 4)


def _sc_agg1_body(xcat_hbm, src_hbm, dst_hbm, out_hbm, degp_hbm,
                  sidx, didx, b0, b1, b2, b3, zbuf, deg, shared,
                  g0, g1, g2, g3, s0, s1, s2, s3):
    c = lax.axis_index("c")
    s = lax.axis_index("s")
    wid = s * NC + c

    pltpu.sync_copy(src_hbm.at[s], sidx)
    pltpu.sync_copy(dst_hbm.at[s], didx)

    off = c * NP

    def adj(i, carry):
        for k in range(RW // 16):
            sidx[i, pl.ds(k * 16, 16)] = sidx[i, pl.ds(k * 16, 16)] + off
        return carry

    lax.fori_loop(0, NRS, adj, 0)

    z16 = jnp.zeros((16,), jnp.float32)

    def dz(i, carry):
        deg[pl.ds(i * 16, 16)] = z16
        return carry

    lax.fori_loop(0, NP // 16, dz, 0)

    one16 = jnp.ones((16,), jnp.float32)

    def count_deg(j):
        for k in range(RW // 16):
            plsc.addupdate_scatter(deg, [didx[j, pl.ds(k * 16, 16)]], one16)

    _zero_shared(zbuf, shared, s, XH)
    plsc.subcore_barrier()
    _edge_pipeline(xcat_hbm, sidx, didx, shared, (b0, b1, b2, b3),
                   (g0, g1, g2, g3), (s0, s1, s2, s3), NRS,
                   extra=count_deg)
    plsc.subcore_barrier()
    pltpu.sync_copy(shared.at[pl.ds(s * NPS, NPS)],
                    out_hbm.at[c, pl.ds(s * NPS, NPS), :])
    pltpu.sync_copy(deg, degp_hbm.at[wid])


@functools.lru_cache(maxsize=None)
def _sc_agg1():
    return pl.kernel(
        _sc_agg1_body,
        out_type=[
            jax.ShapeDtypeStruct((NC, NP, XH), jnp.float32),
            jax.ShapeDtypeStruct((NW, NP), jnp.float32),
        ],
        mesh=_mesh(),
        compiler_params=pltpu.CompilerParams(use_tc_tiling_on_sc=False,
                                             needs_layout_passes=False),
        scratch_types=[
            pltpu.VMEM((NRS, RW), jnp.int32),
            pltpu.VMEM((NRS, RW), jnp.int32),
            pltpu.VMEM((RW, XH), jnp.float32),
            pltpu.VMEM((RW, XH), jnp.float32),
            pltpu.VMEM((RW, XH), jnp.float32),
            pltpu.VMEM((RW, XH), jnp.float32),
            pltpu.VMEM((ZR, XH), jnp.float32),
            pltpu.VMEM((NP,), jnp.float32),
            pltpu.VMEM_SHARED((NP, XH), jnp.float32),
        ] + [pltpu.SemaphoreType.DMA] * 8,
    )


def _sc_layer2_body(g1_hbm, se_hbm, src_hbm, dst_hbm, ids_hbm,
                    ap_hbm, sg_hbm,
                    sidx, didx, b0, b1, b2, b3, zbuf, idv, serows, idv2,
                    aggrows, shared, g0, g1s, g2, g3, s0, s1, s2, s3):
    c = lax.axis_index("c")
    s = lax.axis_index("s")
    wid = s * NC + c

    pltpu.sync_copy(src_hbm.at[wid], sidx)
    pltpu.sync_copy(dst_hbm.at[wid], didx)
    _zero_shared(zbuf, shared, s, OUT)
    plsc.subcore_barrier()
    _edge_pipeline(g1_hbm, sidx, didx, shared, (b0, b1, b2, b3),
                   (g0, g1s, g2, g3), (s0, s1, s2, s3), NR)

    nid = (2 * B) // NW
    pltpu.sync_copy(ids_hbm.at[pl.ds(wid * nid, nid)], idv)
    pltpu.async_copy(se_hbm.at[idv], serows, g0).wait()
    pltpu.sync_copy(serows, sg_hbm.at[pl.ds(wid * nid, nid)])

    plsc.subcore_barrier()

    nid2 = (2 * B) // NS
    pltpu.sync_copy(ids_hbm.at[pl.ds(s * nid2, nid2)], idv2)
    pltpu.async_copy(shared.at[idv2], aggrows, g0).wait()
    pltpu.sync_copy(aggrows, ap_hbm.at[c, pl.ds(s * nid2, nid2), :])


@functools.lru_cache(maxsize=None)
def _sc_layer2():
    return pl.kernel(
        _sc_layer2_body,
        out_type=[
            jax.ShapeDtypeStruct((NC, 2 * B, OUT), jnp.float32),
            jax.ShapeDtypeStruct((2 * B, SE), jnp.float32),
        ],
        mesh=_mesh(),
        compiler_params=pltpu.CompilerParams(use_tc_tiling_on_sc=False,
                                             needs_layout_passes=False),
        scratch_types=[
            pltpu.VMEM((NR, RW), jnp.int32),
            pltpu.VMEM((NR, RW), jnp.int32),
            pltpu.VMEM((RW, OUT), jnp.float32),
            pltpu.VMEM((RW, OUT), jnp.float32),
            pltpu.VMEM((RW, OUT), jnp.float32),
            pltpu.VMEM((RW, OUT), jnp.float32),
            pltpu.VMEM((ZR, OUT), jnp.float32),
            pltpu.VMEM(((2 * B) // NW,), jnp.int32),
            pltpu.VMEM(((2 * B) // NW, SE), jnp.float32),
            pltpu.VMEM(((2 * B) // NS,), jnp.int32),
            pltpu.VMEM(((2 * B) // NS, OUT), jnp.float32),
            pltpu.VMEM_SHARED((NP, OUT), jnp.float32),
        ] + [pltpu.SemaphoreType.DMA] * 8,
    )


_RB = 1024


def _tc_b_kernel(aggp_ref, degp_ref, x_ref, ws1_ref, wn1_ref, b1_ref,
                 ws2_ref, wn2_ref, b2_ref, g1_ref, se_ref):
    a0 = aggp_ref[0]
    a1 = aggp_ref[1]
    deg = jnp.maximum(0.5 * jnp.sum(degp_ref[...], axis=0), 1.0)
    invdeg = 1.0 / deg
    mean = jnp.concatenate([a0, a1], axis=1) * invdeg[:, None]
    h1 = jnp.maximum(
        jnp.dot(x_ref[...], ws1_ref[...], preferred_element_type=jnp.float32)
        + jnp.dot(mean, wn1_ref[...], preferred_element_type=jnp.float32)
        + b1_ref[...][None, :], 0.0)
    g1_ref[...] = jnp.dot(h1, wn2_ref[...], preferred_element_type=jnp.float32)
    sf = jnp.dot(h1, ws2_ref[...], preferred_element_type=jnp.float32) \
        + b2_ref[...][None, :]
    se_ref[...] = jnp.concatenate(
        [sf, invdeg[:, None], jnp.zeros((_RB, SE - OUT - 1), jnp.float32)],
        axis=1)


def _tc_b(aggp, degp, x, ws1, wn1, b1, ws2, wn2, b2):
    return pl.pallas_call(
        _tc_b_kernel,
        grid=(NP // _RB,),
        in_specs=[
            pl.BlockSpec((NC, _RB, XH), lambda i: (0, i, 0)),
            pl.BlockSpec((NW, _RB), lambda i: (0, i)),
            pl.BlockSpec((_RB, D), lambda i: (i, 0)),
            pl.BlockSpec((D, HID), lambda i: (0, 0)),
            pl.BlockSpec((D, HID), lambda i: (0, 0)),
            pl.BlockSpec((HID,), lambda i: (0,)),
            pl.BlockSpec((HID, OUT), lambda i: (0, 0)),
            pl.BlockSpec((HID, OUT), lambda i: (0, 0)),
            pl.BlockSpec((OUT,), lambda i: (0,)),
        ],
        out_specs=[
            pl.BlockSpec((_RB, OUT), lambda i: (i, 0)),
            pl.BlockSpec((_RB, SE), lambda i: (i, 0)),
        ],
        out_shape=[
            jax.ShapeDtypeStruct((NP, OUT), jnp.float32),
            jax.ShapeDtypeStruct((NP, SE), jnp.float32),
        ],
    )(aggp, degp, x, ws1, wn1, b1, ws2, wn2, b2)


def _tc_d_kernel(sg_ref, ap_ref, fcw_ref, fcb_ref, g1_ref, be1_ref,
                 f1w_ref, f1b_ref, g2_ref, be2_ref, f2w_ref, f2b_ref,
                 out_ref):
    agg2 = ap_ref[0] + ap_ref[1]
    slg = sg_ref[:, :OUT] + sg_ref[:, OUT:OUT + 1] * agg2
    fused = jnp.concatenate([slg[:B], slg[B:]], axis=1)

    def bn_relu(h, g, beta):
        mu = jnp.mean(h, axis=0, keepdims=True)
        var = jnp.mean((h - mu) * (h - mu), axis=0, keepdims=True)
        return jnp.maximum(g[None, :] * (h - mu) / jnp.sqrt(var + 1e-5)
                           + beta[None, :], 0.0)

    h = jnp.dot(fused, fcw_ref[...], preferred_element_type=jnp.float32) \
        + fcb_ref[...][None, :]
    h = bn_relu(h, g1_ref[...], be1_ref[...])
    h = jnp.dot(h, f1w_ref[...], preferred_element_type=jnp.float32) \
        + f1b_ref[...][None, :]
    h = bn_relu(h, g2_ref[...], be2_ref[...])
    out_ref[...] = jnp.dot(h, f2w_ref[...],
                           preferred_element_type=jnp.float32) \
        + f2b_ref[...][None, :]


def _tc_d(sg, ap, fc_W, fc_b, bn1_g, bn1_b, fc1_W, fc1_b, bn2_g, bn2_b,
          fc2_W, fc2_b):
    return pl.pallas_call(
        _tc_d_kernel,
        out_shape=jax.ShapeDtypeStruct((B, 1), jnp.float32),
    )(sg, ap, fc_W, fc_b, bn1_g, bn1_b, fc1_W, fc1_b, bn2_g, bn2_b,
      fc2_W, fc2_b)


def kernel(x, edge_index, head_ids, tail_ids,
           W_self1, W_neigh1, b1, W_self2, W_neigh2, b2,
           fc_W, fc_b, bn1_g, bn1_b, fc1_W, fc1_b, bn2_g, bn2_b,
           fc2_W, fc2_b):
    srcf = edge_index[0].astype(jnp.int32)
    dstf = edge_index[1].astype(jnp.int32)
    src = srcf.reshape(NW, NR, RW)
    dst = dstf.reshape(NW, NR, RW)
    src16 = srcf.reshape(NS, NRS, RW)
    dst16 = dstf.reshape(NS, NRS, RW)
    xp = jnp.pad(x, ((0, NP - N), (0, 0)))
    xcat = jnp.concatenate([xp[:, :XH], xp[:, XH:]], axis=0)
    ids2 = jnp.concatenate([head_ids, tail_ids]).astype(jnp.int32)

    aggp, degp = _sc_agg1()(xcat, src16, dst16)
    g1, se = _tc_b(aggp, degp, xp, W_self1, W_neigh1, b1, W_self2, W_neigh2,
                   b2)
    ap, sg = _sc_layer2()(g1, se, src, dst, ids2)
    return _tc_d(sg, ap, fc_W, fc_b, bn1_g, bn1_b, fc1_W, fc1_b,
                 bn2_g, bn2_b, fc2_W, fc2_b)


# speedup vs baseline: 14.8493x; 1.1085x over previous
import functools

import jax
import jax.numpy as jnp
from jax import lax
from jax.experimental import pallas as pl
from jax.experimental.pallas import tpu as pltpu
from jax.experimental.pallas import tpu_sc as plsc

N = 10000
D = 128
E = 320000
B = 1024
HID = 256
OUT = 64

NC = 2
NS = 16
NW = NC * NS

RW = 80
ROWS = E // RW
NR = ROWS // NW
NP = 10240
NPS = NP // NS
ZR = 128

XH = 64
NRS = ROWS // NS
SE = 80

@functools.lru_cache(maxsize=None)
def _mesh():
    return plsc.VectorSubcoreMesh(core_axis_name="c", subcore_axis_name="s",
                                  num_cores=NC, num_subcores=NS)


def _zero_shared(zbuf, shared, s, width):
    z16 = jnp.zeros((16,), jnp.float32)

    def zrow(i, carry):
        for k in range(width // 16):
            zbuf[i, pl.ds(k * 16, 16)] = z16
        return carry

    lax.fori_loop(0, ZR, zrow, 0)
    for k in range(NPS // ZR):
        pltpu.sync_copy(zbuf, shared.at[pl.ds(s * NPS + k * ZR, ZR)])


_PD = 3
_NBUF = 2 * _PD


def _edge_pipeline(table_hbm, sidx, didx, shared, bufs, gsems, ssems, n,
                   extra=None):
    assert n >= 2 * _NBUF

    def startG(r, k):
        pltpu.async_copy(table_hbm.at[sidx.at[r]], bufs[k], gsems[k])

    def waitG(r, k):
        pltpu.make_async_copy(table_hbm.at[sidx.at[r]], bufs[k],
                              gsems[k]).wait()

    def startS(r, k):
        pltpu.async_copy(bufs[k], shared.at[didx.at[r]], ssems[k], add=True)

    def waitS(r, k):
        pltpu.make_async_copy(bufs[k], shared.at[didx.at[r]],
                              ssems[k]).wait()

    def slot(r, k, has_prev, has_next):
        waitG(r, k)
        startS(r, k)
        if has_prev:
            waitS(r - _PD, (k - _PD) % _NBUF)
        if has_next:
            startG(r + _PD, (k + _PD) % _NBUF)
        if extra is not None:
            extra(r)

    for r in range(_PD):
        startG(r, r)
    for r in range(_NBUF):
        slot(r, r, r >= _PD, r + _PD < n)

    M = (n - 2 * _NBUF) // _NBUF

    def body(m, carry):
        base = _NBUF + _NBUF * m
        for k in range(_NBUF):
            slot(base + k, k, True, True)
        return carry

    lax.fori_loop(0, M, body, 0)
    for r in range(_NBUF + _NBUF * M, n):
        slot(r, r % _NBUF, True, r + _PD < n)
    for r in range(n - _PD, n):
        waitS(r, r % _NBUF)


def _sc_agg1_body(xcat_hbm, src_hbm, dst_hbm, out_hbm, degp_hbm,
                  sidx, didx, b0, b1, b2, b3, b4, b5, zbuf, deg, shared,
                  g0, g1, g2, g3, g4, g5, s0, s1, s2, s3, s4, s5):
    c = lax.axis_index("c")
    s = lax.axis_index("s")
    wid = s * NC + c

    pltpu.sync_copy(src_hbm.at[s], sidx)
    pltpu.sync_copy(dst_hbm.at[s], didx)

    off = c * NP

    def adj(i, carry):
        for k in range(RW // 16):
            sidx[i, pl.ds(k * 16, 16)] = sidx[i, pl.ds(k * 16, 16)] + off
        return carry

    lax.fori_loop(0, NRS, adj, 0)

    z16 = jnp.zeros((16,), jnp.float32)

    def dz(i, carry):
        deg[pl.ds(i * 16, 16)] = z16
        return carry

    lax.fori_loop(0, NP // 16, dz, 0)

    one16 = jnp.ones((16,), jnp.float32)

    def count_deg(j):
        for k in range(RW // 16):
            plsc.addupdate_scatter(deg, [didx[j, pl.ds(k * 16, 16)]], one16)

    _zero_shared(zbuf, shared, s, XH)
    plsc.subcore_barrier()
    _edge_pipeline(xcat_hbm, sidx, didx, shared, (b0, b1, b2, b3, b4, b5),
                   (g0, g1, g2, g3, g4, g5), (s0, s1, s2, s3, s4, s5), NRS,
                   extra=count_deg)
    plsc.subcore_barrier()
    pltpu.sync_copy(shared.at[pl.ds(s * NPS, NPS)],
                    out_hbm.at[c, pl.ds(s * NPS, NPS), :])
    pltpu.sync_copy(deg, degp_hbm.at[wid])


@functools.lru_cache(maxsize=None)
def _sc_agg1():
    return pl.kernel(
        _sc_agg1_body,
        out_type=[
            jax.ShapeDtypeStruct((NC, NP, XH), jnp.float32),
            jax.ShapeDtypeStruct((NW, NP), jnp.float32),
        ],
        mesh=_mesh(),
        compiler_params=pltpu.CompilerParams(use_tc_tiling_on_sc=False,
                                             needs_layout_passes=False),
        scratch_types=[
            pltpu.VMEM((NRS, RW), jnp.int32),
            pltpu.VMEM((NRS, RW), jnp.int32),
            pltpu.VMEM((RW, XH), jnp.float32),
            pltpu.VMEM((RW, XH), jnp.float32),
            pltpu.VMEM((RW, XH), jnp.float32),
            pltpu.VMEM((RW, XH), jnp.float32),
            pltpu.VMEM((RW, XH), jnp.float32),
            pltpu.VMEM((RW, XH), jnp.float32),
            pltpu.VMEM((ZR, XH), jnp.float32),
            pltpu.VMEM((NP,), jnp.float32),
            pltpu.VMEM_SHARED((NP, XH), jnp.float32),
        ] + [pltpu.SemaphoreType.DMA] * 12,
    )


def _sc_layer2_body(g1_hbm, se_hbm, src_hbm, dst_hbm, ids_hbm,
                    ap_hbm, sg_hbm,
                    sidx, didx, b0, b1, b2, b3, b4, b5, zbuf, idv, serows,
                    idv2, aggrows, shared, g0, g1s, g2, g3, g4, g5,
                    s0, s1, s2, s3, s4, s5):
    c = lax.axis_index("c")
    s = lax.axis_index("s")
    wid = s * NC + c

    pltpu.sync_copy(src_hbm.at[wid], sidx)
    pltpu.sync_copy(dst_hbm.at[wid], didx)
    _zero_shared(zbuf, shared, s, OUT)
    plsc.subcore_barrier()
    _edge_pipeline(g1_hbm, sidx, didx, shared, (b0, b1, b2, b3, b4, b5),
                   (g0, g1s, g2, g3, g4, g5), (s0, s1, s2, s3, s4, s5), NR)

    nid = (2 * B) // NW
    pltpu.sync_copy(ids_hbm.at[pl.ds(wid * nid, nid)], idv)
    pltpu.async_copy(se_hbm.at[idv], serows, g0).wait()
    pltpu.sync_copy(serows, sg_hbm.at[pl.ds(wid * nid, nid)])

    plsc.subcore_barrier()

    nid2 = (2 * B) // NS
    pltpu.sync_copy(ids_hbm.at[pl.ds(s * nid2, nid2)], idv2)
    pltpu.async_copy(shared.at[idv2], aggrows, g0).wait()
    pltpu.sync_copy(aggrows, ap_hbm.at[c, pl.ds(s * nid2, nid2), :])


@functools.lru_cache(maxsize=None)
def _sc_layer2():
    return pl.kernel(
        _sc_layer2_body,
        out_type=[
            jax.ShapeDtypeStruct((NC, 2 * B, OUT), jnp.float32),
            jax.ShapeDtypeStruct((2 * B, SE), jnp.float32),
        ],
        mesh=_mesh(),
        compiler_params=pltpu.CompilerParams(use_tc_tiling_on_sc=False,
                                             needs_layout_passes=False),
        scratch_types=[
            pltpu.VMEM((NR, RW), jnp.int32),
            pltpu.VMEM((NR, RW), jnp.int32),
            pltpu.VMEM((RW, OUT), jnp.float32),
            pltpu.VMEM((RW, OUT), jnp.float32),
            pltpu.VMEM((RW, OUT), jnp.float32),
            pltpu.VMEM((RW, OUT), jnp.float32),
            pltpu.VMEM((RW, OUT), jnp.float32),
            pltpu.VMEM((RW, OUT), jnp.float32),
            pltpu.VMEM((ZR, OUT), jnp.float32),
            pltpu.VMEM(((2 * B) // NW,), jnp.int32),
            pltpu.VMEM(((2 * B) // NW, SE), jnp.float32),
            pltpu.VMEM(((2 * B) // NS,), jnp.int32),
            pltpu.VMEM(((2 * B) // NS, OUT), jnp.float32),
            pltpu.VMEM_SHARED((NP, OUT), jnp.float32),
        ] + [pltpu.SemaphoreType.DMA] * 12,
    )


_RB = 1024


def _tc_b_kernel(aggp_ref, degp_ref, x_ref, ws1_ref, wn1_ref, b1_ref,
                 ws2_ref, wn2_ref, b2_ref, g1_ref, se_ref):
    a0 = aggp_ref[0]
    a1 = aggp_ref[1]
    deg = jnp.maximum(0.5 * jnp.sum(degp_ref[...], axis=0), 1.0)
    invdeg = 1.0 / deg
    mean = jnp.concatenate([a0, a1], axis=1) * invdeg[:, None]
    h1 = jnp.maximum(
        jnp.dot(x_ref[...], ws1_ref[...], preferred_element_type=jnp.float32)
        + jnp.dot(mean, wn1_ref[...], preferred_element_type=jnp.float32)
        + b1_ref[...][None, :], 0.0)
    g1_ref[...] = jnp.dot(h1, wn2_ref[...], preferred_element_type=jnp.float32)
    sf = jnp.dot(h1, ws2_ref[...], preferred_element_type=jnp.float32) \
        + b2_ref[...][None, :]
    se_ref[...] = jnp.concatenate(
        [sf, invdeg[:, None], jnp.zeros((_RB, SE - OUT - 1), jnp.float32)],
        axis=1)


def _tc_b(aggp, degp, x, ws1, wn1, b1, ws2, wn2, b2):
    return pl.pallas_call(
        _tc_b_kernel,
        grid=(NP // _RB,),
        in_specs=[
            pl.BlockSpec((NC, _RB, XH), lambda i: (0, i, 0)),
            pl.BlockSpec((NW, _RB), lambda i: (0, i)),
            pl.BlockSpec((_RB, D), lambda i: (i, 0)),
            pl.BlockSpec((D, HID), lambda i: (0, 0)),
            pl.BlockSpec((D, HID), lambda i: (0, 0)),
            pl.BlockSpec((HID,), lambda i: (0,)),
            pl.BlockSpec((HID, OUT), lambda i: (0, 0)),
            pl.BlockSpec((HID, OUT), lambda i: (0, 0)),
            pl.BlockSpec((OUT,), lambda i: (0,)),
        ],
        out_specs=[
            pl.BlockSpec((_RB, OUT), lambda i: (i, 0)),
            pl.BlockSpec((_RB, SE), lambda i: (i, 0)),
        ],
        out_shape=[
            jax.ShapeDtypeStruct((NP, OUT), jnp.float32),
            jax.ShapeDtypeStruct((NP, SE), jnp.float32),
        ],
    )(aggp, degp, x, ws1, wn1, b1, ws2, wn2, b2)


def _tc_d_kernel(sg_ref, ap_ref, fcw_ref, fcb_ref, g1_ref, be1_ref,
                 f1w_ref, f1b_ref, g2_ref, be2_ref, f2w_ref, f2b_ref,
                 out_ref):
    agg2 = ap_ref[0] + ap_ref[1]
    slg = sg_ref[:, :OUT] + sg_ref[:, OUT:OUT + 1] * agg2
    fused = jnp.concatenate([slg[:B], slg[B:]], axis=1)

    def bn_relu(h, g, beta):
        mu = jnp.mean(h, axis=0, keepdims=True)
        var = jnp.mean((h - mu) * (h - mu), axis=0, keepdims=True)
        return jnp.maximum(g[None, :] * (h - mu) / jnp.sqrt(var + 1e-5)
                           + beta[None, :], 0.0)

    h = jnp.dot(fused, fcw_ref[...], preferred_element_type=jnp.float32) \
        + fcb_ref[...][None, :]
    h = bn_relu(h, g1_ref[...], be1_ref[...])
    h = jnp.dot(h, f1w_ref[...], preferred_element_type=jnp.float32) \
        + f1b_ref[...][None, :]
    h = bn_relu(h, g2_ref[...], be2_ref[...])
    out_ref[...] = jnp.dot(h, f2w_ref[...],
                           preferred_element_type=jnp.float32) \
        + f2b_ref[...][None, :]


def _tc_d(sg, ap, fc_W, fc_b, bn1_g, bn1_b, fc1_W, fc1_b, bn2_g, bn2_b,
          fc2_W, fc2_b):
    return pl.pallas_call(
        _tc_d_kernel,
        out_shape=jax.ShapeDtypeStruct((B, 1), jnp.float32),
    )(sg, ap, fc_W, fc_b, bn1_g, bn1_b, fc1_W, fc1_b, bn2_g, bn2_b,
      fc2_W, fc2_b)


def kernel(x, edge_index, head_ids, tail_ids,
           W_self1, W_neigh1, b1, W_self2, W_neigh2, b2,
           fc_W, fc_b, bn1_g, bn1_b, fc1_W, fc1_b, bn2_g, bn2_b,
           fc2_W, fc2_b):
    srcf = edge_index[0].astype(jnp.int32)
    dstf = edge_index[1].astype(jnp.int32)
    src = srcf.reshape(NW, NR, RW)
    dst = dstf.reshape(NW, NR, RW)
    src16 = srcf.reshape(NS, NRS, RW)
    dst16 = dstf.reshape(NS, NRS, RW)
    xp = jnp.pad(x, ((0, NP - N), (0, 0)))
    xcat = jnp.concatenate([xp[:, :XH], xp[:, XH:]], axis=0)
    ids2 = jnp.concatenate([head_ids, tail_ids]).astype(jnp.int32)

    aggp, degp = _sc_agg1()(xcat, src16, dst16)
    g1, se = _tc_b(aggp, degp, xp, W_self1, W_neigh1, b1, W_self2, W_neigh2,
                   b2)
    ap, sg = _sc_layer2()(g1, se, src, dst, ids2)
    return _tc_d(sg, ap, fc_W, fc_b, bn1_g, bn1_b, fc1_W, fc1_b,
                 bn2_g, bn2_b, fc2_W, fc2_b)


# speedup vs baseline: 14.8795x; 1.0020x over previous
import functools

import jax
import jax.numpy as jnp
from jax import lax
from jax.experimental import pallas as pl
from jax.experimental.pallas import tpu as pltpu
from jax.experimental.pallas import tpu_sc as plsc

N = 10000
D = 128
E = 320000
B = 1024
HID = 256
OUT = 64

NC = 2
NS = 16
NW = NC * NS

RW = 80
ROWS = E // RW
NR = ROWS // NW
NP = 10240
NPS = NP // NS
ZR = 128

XH = 64
NRS = ROWS // NS
SE = 80

@functools.lru_cache(maxsize=None)
def _mesh():
    return plsc.VectorSubcoreMesh(core_axis_name="c", subcore_axis_name="s",
                                  num_cores=NC, num_subcores=NS)


def _zero_shared(zbuf, shared, s, width):
    z16 = jnp.zeros((16,), jnp.float32)

    def zrow(i, carry):
        for k in range(width // 16):
            zbuf[i, pl.ds(k * 16, 16)] = z16
        return carry

    lax.fori_loop(0, ZR, zrow, 0)
    for k in range(NPS // ZR):
        pltpu.sync_copy(zbuf, shared.at[pl.ds(s * NPS + k * ZR, ZR)])


_PD = 3
_NBUF = 2 * _PD


def _edge_pipeline(table_hbm, sidx, didx, shared, bufs, gsems, ssems, n,
                   extra=None):
    assert n >= 2 * _NBUF

    def startG(r, k):
        pltpu.async_copy(table_hbm.at[sidx.at[r]], bufs[k], gsems[k])

    def waitG(r, k):
        pltpu.make_async_copy(table_hbm.at[sidx.at[r]], bufs[k],
                              gsems[k]).wait()

    def startS(r, k):
        pltpu.async_copy(bufs[k], shared.at[didx.at[r]], ssems[k], add=True)

    def waitS(r, k):
        pltpu.make_async_copy(bufs[k], shared.at[didx.at[r]],
                              ssems[k]).wait()

    def slot(r, k, has_prev, has_next):
        waitG(r, k)
        startS(r, k)
        if has_prev:
            waitS(r - _PD, (k - _PD) % _NBUF)
        if has_next:
            startG(r + _PD, (k + _PD) % _NBUF)
        if extra is not None:
            extra(r)

    for r in range(_PD):
        startG(r, r)
    for r in range(_NBUF):
        slot(r, r, r >= _PD, r + _PD < n)

    M = (n - 2 * _NBUF) // _NBUF

    def body(m, carry):
        base = _NBUF + _NBUF * m
        for k in range(_NBUF):
            slot(base + k, k, True, True)
        return carry

    lax.fori_loop(0, M, body, 0)
    for r in range(_NBUF + _NBUF * M, n):
        slot(r, r % _NBUF, True, r + _PD < n)
    for r in range(n - _PD, n):
        waitS(r, r % _NBUF)


def _sc_agg1_body(xcat_hbm, src_hbm, dst_hbm, out_hbm, degp_hbm,
                  sidx, didx, b0, b1, b2, b3, b4, b5, b6, b7, zbuf, deg,
                  shared, g0, g1, g2, g3, g4, g5, g6, g7,
                  s0, s1, s2, s3, s4, s5, s6, s7):
    c = lax.axis_index("c")
    s = lax.axis_index("s")
    wid = s * NC + c

    pltpu.sync_copy(src_hbm.at[s], sidx)
    pltpu.sync_copy(dst_hbm.at[s], didx)

    off = c * NP

    def adj(i, carry):
        for k in range(RW // 16):
            sidx[i, pl.ds(k * 16, 16)] = sidx[i, pl.ds(k * 16, 16)] + off
        return carry

    lax.fori_loop(0, NRS, adj, 0)

    z16 = jnp.zeros((16,), jnp.float32)

    def dz(i, carry):
        deg[pl.ds(i * 16, 16)] = z16
        return carry

    lax.fori_loop(0, NP // 16, dz, 0)

    one16 = jnp.ones((16,), jnp.float32)

    def count_deg(j):
        for k in range(RW // 16):
            plsc.addupdate_scatter(deg, [didx[j, pl.ds(k * 16, 16)]], one16)

    _zero_shared(zbuf, shared, s, XH)
    plsc.subcore_barrier()
    _edge_pipeline(xcat_hbm, sidx, didx, shared,
                   (b0, b1, b2, b3, b4, b5, b6, b7),
                   (g0, g1, g2, g3, g4, g5, g6, g7),
                   (s0, s1, s2, s3, s4, s5, s6, s7), NRS,
                   extra=count_deg)
    plsc.subcore_barrier()
    pltpu.sync_copy(shared.at[pl.ds(s * NPS, NPS)],
                    out_hbm.at[c, pl.ds(s * NPS, NPS), :])
    pltpu.sync_copy(deg, degp_hbm.at[wid])


@functools.lru_cache(maxsize=None)
def _sc_agg1():
    return pl.kernel(
        _sc_agg1_body,
        out_type=[
            jax.ShapeDtypeStruct((NC, NP, XH), jnp.float32),
            jax.ShapeDtypeStruct((NW, NP), jnp.float32),
        ],
        mesh=_mesh(),
        compiler_params=pltpu.CompilerParams(use_tc_tiling_on_sc=False,
                                             needs_layout_passes=False),
        scratch_types=[
            pltpu.VMEM((NRS, RW), jnp.int32),
            pltpu.VMEM((NRS, RW), jnp.int32),
        ] + [pltpu.VMEM((RW, XH), jnp.float32)] * 8 + [
            pltpu.VMEM((ZR, XH), jnp.float32),
            pltpu.VMEM((NP,), jnp.float32),
            pltpu.VMEM_SHARED((NP, XH), jnp.float32),
        ] + [pltpu.SemaphoreType.DMA] * 16,
    )


def _sc_layer2_body(g1_hbm, se_hbm, src_hbm, dst_hbm, ids_hbm,
                    ap_hbm, sg_hbm,
                    sidx, didx, b0, b1, b2, b3, b4, b5, b6, b7, zbuf, idv,
                    serows, idv2, aggrows, shared, g0, g1s, g2, g3, g4, g5,
                    g6, g7, s0, s1, s2, s3, s4, s5, s6, s7):
    c = lax.axis_index("c")
    s = lax.axis_index("s")
    wid = s * NC + c

    pltpu.sync_copy(src_hbm.at[wid], sidx)
    pltpu.sync_copy(dst_hbm.at[wid], didx)
    _zero_shared(zbuf, shared, s, OUT)
    plsc.subcore_barrier()
    _edge_pipeline(g1_hbm, sidx, didx, shared,
                   (b0, b1, b2, b3, b4, b5, b6, b7),
                   (g0, g1s, g2, g3, g4, g5, g6, g7),
                   (s0, s1, s2, s3, s4, s5, s6, s7), NR)

    nid = (2 * B) // NW
    pltpu.sync_copy(ids_hbm.at[pl.ds(wid * nid, nid)], idv)
    pltpu.async_copy(se_hbm.at[idv], serows, g0).wait()
    pltpu.sync_copy(serows, sg_hbm.at[pl.ds(wid * nid, nid)])

    plsc.subcore_barrier()

    nid2 = (2 * B) // NS
    pltpu.sync_copy(ids_hbm.at[pl.ds(s * nid2, nid2)], idv2)
    pltpu.async_copy(shared.at[idv2], aggrows, g0).wait()
    pltpu.sync_copy(aggrows, ap_hbm.at[c, pl.ds(s * nid2, nid2), :])


@functools.lru_cache(maxsize=None)
def _sc_layer2():
    return pl.kernel(
        _sc_layer2_body,
        out_type=[
            jax.ShapeDtypeStruct((NC, 2 * B, OUT), jnp.float32),
            jax.ShapeDtypeStruct((2 * B, SE), jnp.float32),
        ],
        mesh=_mesh(),
        compiler_params=pltpu.CompilerParams(use_tc_tiling_on_sc=False,
                                             needs_layout_passes=False),
        scratch_types=[
            pltpu.VMEM((NR, RW), jnp.int32),
            pltpu.VMEM((NR, RW), jnp.int32),
        ] + [pltpu.VMEM((RW, OUT), jnp.float32)] * 8 + [
            pltpu.VMEM((ZR, OUT), jnp.float32),
            pltpu.VMEM(((2 * B) // NW,), jnp.int32),
            pltpu.VMEM(((2 * B) // NW, SE), jnp.float32),
            pltpu.VMEM(((2 * B) // NS,), jnp.int32),
            pltpu.VMEM(((2 * B) // NS, OUT), jnp.float32),
            pltpu.VMEM_SHARED((NP, OUT), jnp.float32),
        ] + [pltpu.SemaphoreType.DMA] * 16,
    )


_RB = 1024


def _tc_b_kernel(aggp_ref, degp_ref, x_ref, ws1_ref, wn1_ref, b1_ref,
                 ws2_ref, wn2_ref, b2_ref, g1_ref, se_ref):
    a0 = aggp_ref[0]
    a1 = aggp_ref[1]
    deg = jnp.maximum(0.5 * jnp.sum(degp_ref[...], axis=0), 1.0)
    invdeg = 1.0 / deg
    mean = jnp.concatenate([a0, a1], axis=1) * invdeg[:, None]
    h1 = jnp.maximum(
        jnp.dot(x_ref[...], ws1_ref[...], preferred_element_type=jnp.float32)
        + jnp.dot(mean, wn1_ref[...], preferred_element_type=jnp.float32)
        + b1_ref[...][None, :], 0.0)
    g1_ref[...] = jnp.dot(h1, wn2_ref[...], preferred_element_type=jnp.float32)
    sf = jnp.dot(h1, ws2_ref[...], preferred_element_type=jnp.float32) \
        + b2_ref[...][None, :]
    se_ref[...] = jnp.concatenate(
        [sf, invdeg[:, None], jnp.zeros((_RB, SE - OUT - 1), jnp.float32)],
        axis=1)


def _tc_b(aggp, degp, x, ws1, wn1, b1, ws2, wn2, b2):
    return pl.pallas_call(
        _tc_b_kernel,
        grid=(NP // _RB,),
        in_specs=[
            pl.BlockSpec((NC, _RB, XH), lambda i: (0, i, 0)),
            pl.BlockSpec((NW, _RB), lambda i: (0, i)),
            pl.BlockSpec((_RB, D), lambda i: (i, 0)),
            pl.BlockSpec((D, HID), lambda i: (0, 0)),
            pl.BlockSpec((D, HID), lambda i: (0, 0)),
            pl.BlockSpec((HID,), lambda i: (0,)),
            pl.BlockSpec((HID, OUT), lambda i: (0, 0)),
            pl.BlockSpec((HID, OUT), lambda i: (0, 0)),
            pl.BlockSpec((OUT,), lambda i: (0,)),
        ],
        out_specs=[
            pl.BlockSpec((_RB, OUT), lambda i: (i, 0)),
            pl.BlockSpec((_RB, SE), lambda i: (i, 0)),
        ],
        out_shape=[
            jax.ShapeDtypeStruct((NP, OUT), jnp.float32),
            jax.ShapeDtypeStruct((NP, SE), jnp.float32),
        ],
    )(aggp, degp, x, ws1, wn1, b1, ws2, wn2, b2)


def _tc_d_kernel(sg_ref, ap_ref, fcw_ref, fcb_ref, g1_ref, be1_ref,
                 f1w_ref, f1b_ref, g2_ref, be2_ref, f2w_ref, f2b_ref,
                 out_ref):
    agg2 = ap_ref[0] + ap_ref[1]
    slg = sg_ref[:, :OUT] + sg_ref[:, OUT:OUT + 1] * agg2
    fused = jnp.concatenate([slg[:B], slg[B:]], axis=1)

    def bn_relu(h, g, beta):
        mu = jnp.mean(h, axis=0, keepdims=True)
        var = jnp.mean((h - mu) * (h - mu), axis=0, keepdims=True)
        return jnp.maximum(g[None, :] * (h - mu) / jnp.sqrt(var + 1e-5)
                           + beta[None, :], 0.0)

    h = jnp.dot(fused, fcw_ref[...], preferred_element_type=jnp.float32) \
        + fcb_ref[...][None, :]
    h = bn_relu(h, g1_ref[...], be1_ref[...])
    h = jnp.dot(h, f1w_ref[...], preferred_element_type=jnp.float32) \
        + f1b_ref[...][None, :]
    h = bn_relu(h, g2_ref[...], be2_ref[...])
    out_ref[...] = jnp.dot(h, f2w_ref[...],
                           preferred_element_type=jnp.float32) \
        + f2b_ref[...][None, :]


def _tc_d(sg, ap, fc_W, fc_b, bn1_g, bn1_b, fc1_W, fc1_b, bn2_g, bn2_b,
          fc2_W, fc2_b):
    return pl.pallas_call(
        _tc_d_kernel,
        out_shape=jax.ShapeDtypeStruct((B, 1), jnp.float32),
    )(sg, ap, fc_W, fc_b, bn1_g, bn1_b, fc1_W, fc1_b, bn2_g, bn2_b,
      fc2_W, fc2_b)


def kernel(x, edge_index, head_ids, tail_ids,
           W_self1, W_neigh1, b1, W_self2, W_neigh2, b2,
           fc_W, fc_b, bn1_g, bn1_b, fc1_W, fc1_b, bn2_g, bn2_b,
           fc2_W, fc2_b):
    srcf = edge_index[0].astype(jnp.int32)
    dstf = edge_index[1].astype(jnp.int32)
    src = srcf.reshape(NW, NR, RW)
    dst = dstf.reshape(NW, NR, RW)
    src16 = srcf.reshape(NS, NRS, RW)
    dst16 = dstf.reshape(NS, NRS, RW)
    xp = jnp.pad(x, ((0, NP - N), (0, 0)))
    xcat = jnp.concatenate([xp[:, :XH], xp[:, XH:]], axis=0)
    ids2 = jnp.concatenate([head_ids, tail_ids]).astype(jnp.int32)

    aggp, degp = _sc_agg1()(xcat, src16, dst16)
    g1, se = _tc_b(aggp, degp, xp, W_self1, W_neigh1, b1, W_self2, W_neigh2,
                   b2)
    ap, sg = _sc_layer2()(g1, se, src, dst, ids2)
    return _tc_d(sg, ap, fc_W, fc_b, bn1_g, bn1_b, fc1_W, fc1_b,
                 bn2_g, bn2_b, fc2_W, fc2_b)
